# baseline (speedup 1.0000x reference)
import functools

import jax
import jax.numpy as jnp
from jax import lax
from jax.experimental import pallas as pl
from jax.experimental.pallas import tpu as pltpu
from jax.experimental.pallas import tpu_sc as plsc

B, F, T = 4096, 128, 100
N = B * F
NC, NS, L = 2, 16, 16
NW = NC * NS
ROWS_W = N // NW
PB = 8
GR = PB * F
NG = ROWS_W // GR
TSUB = 25
NT = T // TSUB
NCHUNK = NG * NT
CW = TSUB * GR
JG = GR // L


def _spike_times(xv):
    xn = jnp.minimum(jnp.maximum((xv + 1.0) * 0.5, 0.0), 1.0)
    v = (1.0 - xn) * 99.0
    fv = v + 0.5
    ti = fv.astype(jnp.int32)
    tie = ti.astype(jnp.float32) == fv
    odd = (ti & 1) == 1
    ti = ti - jnp.where(tie & odd, 1, 0)
    return jnp.minimum(jnp.maximum(ti, 0), T - 1)


@functools.partial(
    pl.kernel,
    out_type=jax.ShapeDtypeStruct((T, B, F), jnp.float32),
    mesh=plsc.VectorSubcoreMesh(core_axis_name="c", subcore_axis_name="s"),
    compiler_params=pltpu.CompilerParams(needs_layout_passes=False),
    scratch_types=[
        pltpu.VMEM((ROWS_W,), jnp.float32),
        pltpu.VMEM((TSUB, PB, F), jnp.float32),
        pltpu.VMEM((TSUB, PB, F), jnp.float32),
        pltpu.VMEM((GR,), jnp.int32),
        pltpu.VMEM((GR,), jnp.int32),
        pltpu.SemaphoreType.DMA,
        pltpu.SemaphoreType.DMA,
        pltpu.SemaphoreType.DMA,
    ],
)
def _encode(x_hbm, z_hbm, out_hbm, xbuf, ob0, ob1, tb0, tb1, sem0, sem1, semx):
    wid = lax.axis_index("s") * NC + lax.axis_index("c")
    row0 = wid * ROWS_W
    plane0 = wid * (B // NW)
    z0 = pltpu.async_copy(z_hbm, ob0, sem0)
    z1 = pltpu.async_copy(z_hbm, ob1, sem1)
    xcopy = pltpu.async_copy(x_hbm.at[pl.ds(row0, ROWS_W)], xbuf, semx)

    zeros = jnp.zeros((L,), jnp.float32)
    ones = jnp.full((L,), 1.0, jnp.float32)
    lanes = lax.iota(jnp.int32, L)

    z0.wait()
    z1.wait()
    xcopy.wait()

    def _scatter_pass(ob, tb, t0, val):
        def body(j, _):
            ti = tb[pl.ds(j * L, L)]
            m = (ti >= t0) & (ti < t0 + TSUB)
            dt = jnp.minimum(jnp.maximum(ti - t0, 0), TSUB - 1)
            idx_b = jnp.full((L,), 0, jnp.int32) + j // PB
            idx_f = (j % PB) * L + lanes
            plsc.store_scatter(ob, [dt, idx_b, idx_f], val, mask=m)
            return 0

        lax.fori_loop(0, JG, body, 0)

    obufs, tbufs, sems = (ob0, ob1), (tb0, tb1), (sem0, sem1)
    copies = [None] * NCHUNK
    for k in range(NCHUNK):
        g, c = k // NT, k % NT
        p = k & 1
        ob, tb = obufs[p], tbufs[g & 1]
        if c == 0:
            def _times(j, _, tb=tb, g=g):
                xv = xbuf[pl.ds(g * GR + j * L, L)]
                tb[pl.ds(j * L, L)] = _spike_times(xv)
                return 0

            lax.fori_loop(0, JG, _times, 0)
        if k >= 2:
            copies[k - 2].wait()
            g2, c2 = (k - 2) // NT, (k - 2) % NT
            _scatter_pass(ob, tbufs[g2 & 1], c2 * TSUB, zeros)
        _scatter_pass(ob, tb, c * TSUB, ones)
        dst = out_hbm.at[pl.ds(c * TSUB, TSUB), pl.ds(plane0 + g * PB, PB), :]
        copies[k] = pltpu.async_copy(ob, dst, sems[p])

    copies[NCHUNK - 2].wait()
    copies[NCHUNK - 1].wait()


def kernel(x):
    z = jnp.zeros((TSUB, PB, F), jnp.float32)
    return jnp.transpose(_encode(x.reshape(N), z), (1, 2, 0))


# speedup vs baseline: 1.0295x; 1.0295x over previous
import functools

import jax
import jax.numpy as jnp
from jax import lax
from jax.experimental import pallas as pl
from jax.experimental.pallas import tpu as pltpu
from jax.experimental.pallas import tpu_sc as plsc

B, F, T = 4096, 128, 100
N = B * F
NC, NS, L = 2, 16, 16
NW = NC * NS
ROWS_W = N // NW
PB = 8
GR = PB * F
NG = ROWS_W // GR
TSUB = 25
NT = T // TSUB
NCHUNK = NG * NT
CW = TSUB * GR
JG = GR // L


def _spike_times(xv):
    xn = jnp.minimum(jnp.maximum((xv + 1.0) * 0.5, 0.0), 1.0)
    v = (1.0 - xn) * 99.0
    fv = v + 0.5
    ti = fv.astype(jnp.int32)
    tie = ti.astype(jnp.float32) == fv
    odd = (ti & 1) == 1
    ti = ti - jnp.where(tie & odd, 1, 0)
    return jnp.minimum(jnp.maximum(ti, 0), T - 1)


@functools.partial(
    pl.kernel,
    out_type=jax.ShapeDtypeStruct((T, B, F), jnp.float32),
    mesh=plsc.VectorSubcoreMesh(core_axis_name="c", subcore_axis_name="s"),
    compiler_params=pltpu.CompilerParams(needs_layout_passes=False),
    scratch_types=[
        pltpu.VMEM((ROWS_W,), jnp.float32),
        pltpu.VMEM((TSUB, PB, F), jnp.float32),
        pltpu.VMEM((TSUB, PB, F), jnp.float32),
        pltpu.VMEM((GR,), jnp.int32),
        pltpu.VMEM((GR,), jnp.int32),
        pltpu.SemaphoreType.DMA,
        pltpu.SemaphoreType.DMA,
    ],
)
def _encode(x_hbm, out_hbm, xbuf, ob0, ob1, tb0, tb1, sem0, sem1):
    wid = lax.axis_index("s") * NC + lax.axis_index("c")
    row0 = wid * ROWS_W
    plane0 = wid * (B // NW)
    pltpu.sync_copy(x_hbm.at[pl.ds(row0, ROWS_W)], xbuf)

    zeros = jnp.zeros((L,), jnp.float32)
    ones = jnp.full((L,), 1.0, jnp.float32)
    lanes = lax.iota(jnp.int32, L)

    def _zero_init(i, _):
        q = i * L + lanes
        qt, qr = q // (PB * F), q % (PB * F)
        plsc.store_scatter(ob0, [qt, qr // F, qr % F], zeros)
        plsc.store_scatter(ob1, [qt, qr // F, qr % F], zeros)
        return 0

    lax.fori_loop(0, CW // L, _zero_init, 0)

    def _scatter_pass(ob, tb, t0, val):
        def body(j, _):
            ti = tb[pl.ds(j * L, L)]
            m = (ti >= t0) & (ti < t0 + TSUB)
            dt = jnp.minimum(jnp.maximum(ti - t0, 0), TSUB - 1)
            idx_b = jnp.full((L,), 0, jnp.int32) + j // PB
            idx_f = (j % PB) * L + lanes
            plsc.store_scatter(ob, [dt, idx_b, idx_f], val, mask=m)
            return 0

        lax.fori_loop(0, JG, body, 0)

    obufs, tbufs, sems = (ob0, ob1), (tb0, tb1), (sem0, sem1)
    copies = [None] * NCHUNK
    for k in range(NCHUNK):
        g, c = k // NT, k % NT
        p = k & 1
        ob, tb = obufs[p], tbufs[g & 1]
        if c == 0:
            def _times(j, _, tb=tb, g=g):
                xv = xbuf[pl.ds(g * GR + j * L, L)]
                tb[pl.ds(j * L, L)] = _spike_times(xv)
                return 0

            lax.fori_loop(0, JG, _times, 0)
        if k >= 2:
            copies[k - 2].wait()
            g2, c2 = (k - 2) // NT, (k - 2) % NT
            _scatter_pass(ob, tbufs[g2 & 1], c2 * TSUB, zeros)
        _scatter_pass(ob, tb, c * TSUB, ones)
        dst = out_hbm.at[pl.ds(c * TSUB, TSUB), pl.ds(plane0 + g * PB, PB), :]
        copies[k] = pltpu.async_copy(ob, dst, sems[p])

    copies[NCHUNK - 2].wait()
    copies[NCHUNK - 1].wait()


def kernel(x):
    return jnp.transpose(_encode(x.reshape(N)), (1, 2, 0))


# speedup vs baseline: 1.0378x; 1.0081x over previous
import functools

import jax
import jax.numpy as jnp
from jax import lax
from jax.experimental import pallas as pl
from jax.experimental.pallas import tpu as pltpu
from jax.experimental.pallas import tpu_sc as plsc

B, F, T = 4096, 128, 100
N = B * F
NC, NS, L = 2, 16, 16
NW = NC * NS
ROWS_W = N // NW
PB = 8
GR = PB * F
NG = ROWS_W // GR
TSUB = 25
NT = T // TSUB
NCHUNK = NG * NT
CW = TSUB * GR
JG = GR // L


def _spike_times(xv):
    xn = jnp.minimum(jnp.maximum((xv + 1.0) * 0.5, 0.0), 1.0)
    v = (1.0 - xn) * 99.0
    fv = v + 0.5
    ti = fv.astype(jnp.int32)
    tie = ti.astype(jnp.float32) == fv
    odd = (ti & 1) == 1
    ti = ti - jnp.where(tie & odd, 1, 0)
    return jnp.minimum(jnp.maximum(ti, 0), T - 1)


@functools.partial(
    pl.kernel,
    out_type=jax.ShapeDtypeStruct((T, B, F), jnp.float32),
    mesh=plsc.VectorSubcoreMesh(core_axis_name="c", subcore_axis_name="s"),
    compiler_params=pltpu.CompilerParams(needs_layout_passes=False),
    scratch_types=[
        pltpu.VMEM((ROWS_W,), jnp.float32),
        pltpu.VMEM((TSUB, PB, F), jnp.float32),
        pltpu.VMEM((TSUB, PB, F), jnp.float32),
        pltpu.VMEM((TSUB, PB, F), jnp.float32),
        pltpu.VMEM((GR,), jnp.int32),
        pltpu.VMEM((GR,), jnp.int32),
        pltpu.SemaphoreType.DMA,
        pltpu.SemaphoreType.DMA,
        pltpu.SemaphoreType.DMA,
    ],
)
def _encode(x_hbm, out_hbm, xbuf, ob0, ob1, ob2, tb0, tb1, sem0, sem1, sem2):
    wid = lax.axis_index("s") * NC + lax.axis_index("c")
    row0 = wid * ROWS_W
    plane0 = wid * (B // NW)
    pltpu.sync_copy(x_hbm.at[pl.ds(row0, ROWS_W)], xbuf)

    zeros = jnp.zeros((L,), jnp.float32)
    ones = jnp.full((L,), 1.0, jnp.float32)
    lanes = lax.iota(jnp.int32, L)

    def _zero_init(i, _):
        q = i * L + lanes
        qt, qr = q // (PB * F), q % (PB * F)
        plsc.store_scatter(ob0, [qt, qr // F, qr % F], zeros)
        plsc.store_scatter(ob1, [qt, qr // F, qr % F], zeros)
        plsc.store_scatter(ob2, [qt, qr // F, qr % F], zeros)
        return 0

    lax.fori_loop(0, CW // L, _zero_init, 0)

    def _scatter_pass(ob, tb, t0, val):
        def body(j, _):
            ti = tb[pl.ds(j * L, L)]
            m = (ti >= t0) & (ti < t0 + TSUB)
            dt = jnp.minimum(jnp.maximum(ti - t0, 0), TSUB - 1)
            idx_b = jnp.full((L,), 0, jnp.int32) + j // PB
            idx_f = (j % PB) * L + lanes
            plsc.store_scatter(ob, [dt, idx_b, idx_f], val, mask=m)
            return 0

        lax.fori_loop(0, JG, body, 0)

    obufs, tbufs, sems = (ob0, ob1, ob2), (tb0, tb1), (sem0, sem1, sem2)
    NBUF = 3
    copies = [None] * NCHUNK
    for k in range(NCHUNK):
        g, c = k // NT, k % NT
        p = k % NBUF
        ob, tb = obufs[p], tbufs[g & 1]
        if c == 0:
            def _times(j, _, tb=tb, g=g):
                xv = xbuf[pl.ds(g * GR + j * L, L)]
                tb[pl.ds(j * L, L)] = _spike_times(xv)
                return 0

            lax.fori_loop(0, JG, _times, 0)
        if k >= NBUF:
            copies[k - NBUF].wait()
            g2, c2 = (k - NBUF) // NT, (k - NBUF) % NT
            _scatter_pass(ob, tbufs[g2 & 1], c2 * TSUB, zeros)
        _scatter_pass(ob, tb, c * TSUB, ones)
        dst = out_hbm.at[pl.ds(c * TSUB, TSUB), pl.ds(plane0 + g * PB, PB), :]
        copies[k] = pltpu.async_copy(ob, dst, sems[p])

    for k in range(NCHUNK - NBUF, NCHUNK):
        copies[k].wait()


def kernel(x):
    return jnp.transpose(_encode(x.reshape(N)), (1, 2, 0))


# speedup vs baseline: 1.0398x; 1.0019x over previous
import functools

import jax
import jax.numpy as jnp
from jax import lax
from jax.experimental import pallas as pl
from jax.experimental.pallas import tpu as pltpu
from jax.experimental.pallas import tpu_sc as plsc

B, F, T = 4096, 128, 100
N = B * F
NC, NS, L = 2, 16, 16
NW = NC * NS
ROWS_W = N // NW
PB = 8
GR = PB * F
NG = ROWS_W // GR
TSUB = 25
NT = T // TSUB
NCHUNK = NG * NT
CW = TSUB * GR
JG = GR // L


def _spike_times(xv):
    xn = jnp.minimum(jnp.maximum((xv + 1.0) * 0.5, 0.0), 1.0)
    v = (1.0 - xn) * 99.0
    fv = v + 0.5
    ti = fv.astype(jnp.int32)
    tie = ti.astype(jnp.float32) == fv
    odd = (ti & 1) == 1
    ti = ti - jnp.where(tie & odd, 1, 0)
    return jnp.minimum(jnp.maximum(ti, 0), T - 1)


@functools.partial(
    pl.kernel,
    out_type=jax.ShapeDtypeStruct((T, B, F), jnp.float32),
    mesh=plsc.VectorSubcoreMesh(core_axis_name="c", subcore_axis_name="s"),
    compiler_params=pltpu.CompilerParams(needs_layout_passes=False),
    scratch_types=[
        pltpu.VMEM((ROWS_W,), jnp.float32),
        pltpu.VMEM((TSUB, PB, F), jnp.float32),
        pltpu.VMEM((TSUB, PB, F), jnp.float32),
        pltpu.VMEM((TSUB, PB, F), jnp.float32),
        pltpu.VMEM((TSUB, PB, F), jnp.float32),
        pltpu.VMEM((GR,), jnp.int32),
        pltpu.VMEM((GR,), jnp.int32),
        pltpu.SemaphoreType.DMA,
        pltpu.SemaphoreType.DMA,
        pltpu.SemaphoreType.DMA,
        pltpu.SemaphoreType.DMA,
    ],
)
def _encode(x_hbm, out_hbm, xbuf, ob0, ob1, ob2, ob3, tb0, tb1, sem0, sem1, sem2, sem3):
    wid = lax.axis_index("s") * NC + lax.axis_index("c")
    row0 = wid * ROWS_W
    plane0 = wid * (B // NW)
    pltpu.sync_copy(x_hbm.at[pl.ds(row0, ROWS_W)], xbuf)

    zeros = jnp.zeros((L,), jnp.float32)
    ones = jnp.full((L,), 1.0, jnp.float32)
    lanes = lax.iota(jnp.int32, L)

    def _zero_init(i, _):
        q = i * L + lanes
        qt, qr = q // (PB * F), q % (PB * F)
        plsc.store_scatter(ob0, [qt, qr // F, qr % F], zeros)
        plsc.store_scatter(ob1, [qt, qr // F, qr % F], zeros)
        plsc.store_scatter(ob2, [qt, qr // F, qr % F], zeros)
        plsc.store_scatter(ob3, [qt, qr // F, qr % F], zeros)
        return 0

    lax.fori_loop(0, CW // L, _zero_init, 0)

    def _scatter_pass(ob, tb, t0, val):
        def body(j, _):
            ti = tb[pl.ds(j * L, L)]
            m = (ti >= t0) & (ti < t0 + TSUB)
            dt = jnp.minimum(jnp.maximum(ti - t0, 0), TSUB - 1)
            idx_b = jnp.full((L,), 0, jnp.int32) + j // PB
            idx_f = (j % PB) * L + lanes
            plsc.store_scatter(ob, [dt, idx_b, idx_f], val, mask=m)
            return 0

        lax.fori_loop(0, JG, body, 0)

    obufs, tbufs, sems = (ob0, ob1, ob2, ob3), (tb0, tb1), (sem0, sem1, sem2, sem3)
    NBUF = 4
    copies = [None] * NCHUNK
    for k in range(NCHUNK):
        g, c = k // NT, k % NT
        p = k % NBUF
        ob, tb = obufs[p], tbufs[g & 1]
        if c == 0:
            def _times(j, _, tb=tb, g=g):
                xv = xbuf[pl.ds(g * GR + j * L, L)]
                tb[pl.ds(j * L, L)] = _spike_times(xv)
                return 0

            lax.fori_loop(0, JG, _times, 0)
        if k >= NBUF:
            copies[k - NBUF].wait()
            g2, c2 = (k - NBUF) // NT, (k - NBUF) % NT
            _scatter_pass(ob, tbufs[g2 & 1], c2 * TSUB, zeros)
        _scatter_pass(ob, tb, c * TSUB, ones)
        dst = out_hbm.at[pl.ds(c * TSUB, TSUB), pl.ds(plane0 + g * PB, PB), :]
        copies[k] = pltpu.async_copy(ob, dst, sems[p])

    for k in range(NCHUNK - NBUF, NCHUNK):
        copies[k].wait()


def kernel(x):
    return jnp.transpose(_encode(x.reshape(N)), (1, 2, 0))


# speedup vs baseline: 1.0957x; 1.0538x over previous
import functools

import jax
import jax.numpy as jnp
from jax import lax
from jax.experimental import pallas as pl
from jax.experimental.pallas import tpu as pltpu
from jax.experimental.pallas import tpu_sc as plsc

B, F, T = 4096, 128, 100
N = B * F
NC, NS, L = 2, 16, 16
NW = NC * NS
ROWS_W = N // NW
PB = 8
GR = PB * F
NG = ROWS_W // GR
TSUB = 25
NT = T // TSUB
NCHUNK = NG * NT
CW = TSUB * GR
JG = GR // L


def _spike_times(xv):
    xn = jnp.minimum(jnp.maximum((xv + 1.0) * 0.5, 0.0), 1.0)
    v = (1.0 - xn) * 99.0
    fv = v + 0.5
    ti = fv.astype(jnp.int32)
    tie = ti.astype(jnp.float32) == fv
    odd = (ti & 1) == 1
    ti = ti - jnp.where(tie & odd, 1, 0)
    return jnp.minimum(jnp.maximum(ti, 0), T - 1)


@functools.partial(
    pl.kernel,
    out_type=jax.ShapeDtypeStruct((T, B, F), jnp.float32),
    mesh=plsc.VectorSubcoreMesh(core_axis_name="c", subcore_axis_name="s"),
    compiler_params=pltpu.CompilerParams(needs_layout_passes=False),
    scratch_types=[
        pltpu.VMEM((ROWS_W,), jnp.float32),
        pltpu.VMEM((TSUB, PB, F), jnp.float32),
        pltpu.VMEM((TSUB, PB, F), jnp.float32),
        pltpu.VMEM((TSUB, PB, F), jnp.float32),
        pltpu.VMEM((TSUB, PB, F), jnp.float32),
        pltpu.VMEM((GR,), jnp.int32),
        pltpu.VMEM((GR,), jnp.int32),
        pltpu.SemaphoreType.DMA,
        pltpu.SemaphoreType.DMA,
        pltpu.SemaphoreType.DMA,
        pltpu.SemaphoreType.DMA,
    ],
)
def _encode(x_hbm, out_hbm, xbuf, ob0, ob1, ob2, ob3, tb0, tb1, sem0, sem1, sem2, sem3):
    wid = lax.axis_index("s") * NC + lax.axis_index("c")
    row0 = wid * ROWS_W
    plane0 = wid * (B // NW)
    pltpu.sync_copy(x_hbm.at[pl.ds(row0, ROWS_W)], xbuf)

    zeros = jnp.zeros((L,), jnp.float32)
    ones = jnp.full((L,), 1.0, jnp.float32)
    lanes = lax.iota(jnp.int32, L)

    def _zero_init(i, _):
        q = i * L + lanes
        qt, qr = q // (PB * F), q % (PB * F)
        plsc.store_scatter(ob0, [qt, qr // F, qr % F], zeros)
        plsc.store_scatter(ob1, [qt, qr // F, qr % F], zeros)
        plsc.store_scatter(ob2, [qt, qr // F, qr % F], zeros)
        plsc.store_scatter(ob3, [qt, qr // F, qr % F], zeros)
        return 0

    lax.fori_loop(0, CW // L, _zero_init, 0)

    def _scatter_pass(ob, tb, t0, val):
        def body(j, _):
            ti = tb[pl.ds(j * L, L)]
            m = (ti >= t0) & (ti < t0 + TSUB)
            dt = jnp.minimum(jnp.maximum(ti - t0, 0), TSUB - 1)
            idx_b = jnp.full((L,), 0, jnp.int32) + j // PB
            idx_f = (j % PB) * L + lanes
            plsc.store_scatter(ob, [dt, idx_b, idx_f], val, mask=m)
            return 0

        lax.fori_loop(0, JG, body, 0)

    obufs, tbufs, sems = (ob0, ob1, ob2, ob3), (tb0, tb1), (sem0, sem1, sem2, sem3)
    NBUF = 4

    def _times(tb, g):
        def body(j, _):
            xv = xbuf[pl.ds(g * GR + j * L, L)]
            tb[pl.ds(j * L, L)] = _spike_times(xv)
            return 0

        lax.fori_loop(0, JG, body, 0)

    def _dst(g, c):
        gp = pl.multiple_of(plane0 + g * PB, PB)
        return out_hbm.at[pl.ds(c * TSUB, TSUB), pl.ds(gp, PB), :]

    def _super(m, first):
        g0 = 2 * m
        for half, (tb, tbo) in enumerate(((tb0, tb1), (tb1, tb0))):
            g = g0 + half
            _times(tb, g)
            for c in range(NT):
                ob = obufs[c]
                if first and half == 0:
                    pass
                else:
                    pltpu.make_async_copy(ob, _dst(g - 1, c), sems[c]).wait()
                    _scatter_pass(ob, tbo, c * TSUB, zeros)
                _scatter_pass(ob, tb, c * TSUB, ones)
                pltpu.async_copy(ob, _dst(g, c), sems[c])

    _super(0, True)

    def _loop(m, _):
        _super(m, False)
        return 0

    lax.fori_loop(1, NG // 2, _loop, 0)

    g_last = NG - 1
    for c in range(NT):
        pltpu.make_async_copy(obufs[c], _dst(g_last, c), sems[c]).wait()


def kernel(x):
    return jnp.transpose(_encode(x.reshape(N)), (1, 2, 0))
